# baseline (speedup 1.0000x reference)
import functools

import jax
import jax.numpy as jnp
from jax import lax
from jax.experimental import pallas as pl
from jax.experimental.pallas import tpu as pltpu
from jax.experimental.pallas import tpu_sc as plsc

N = 10000
E = 320000
D = 128
H = 128
O = 64
G = 128

NC = 2
NS = 16
NW = NC * NS
CH = 80
EPW = E // NW
NCHUNK = EPW // CH
RPT = 624
RTAIL = N - NS * RPT

CHA = 128
EPWP = 10240
NCHA = EPWP // CHA
NPAD = EPWP - EPW
NA = N + 16

BN = 2000
NB = N // BN

_mesh = plsc.VectorSubcoreMesh(core_axis_name="c", subcore_axis_name="s")


NBUF = 3
NPRE = NBUF - 1
NIDX = NBUF + 1


@functools.partial(
    pl.kernel,
    mesh=_mesh,
    out_type=jax.ShapeDtypeStruct((NC, N, H), jnp.float32),
    scratch_types=[
        pltpu.VMEM((NIDX, 2, CHA), jnp.int32),
        pltpu.VMEM((NBUF, CHA, H), jnp.float32),
        pltpu.VMEM_SHARED((NA, H), jnp.float32),
        pltpu.SemaphoreType.DMA((NIDX,)),
        pltpu.SemaphoreType.DMA((NBUF,)),
        pltpu.SemaphoreType.DMA((NBUF,)),
    ],
)
def _sc_aggregate(hp_hbm, edges_hbm, zeros_hbm, out_hbm,
                  idx_v, rows_v, acc, sem_i, sem_g, sem_s):
    c = lax.axis_index("c")
    s = lax.axis_index("s")
    wid = s * NC + c
    pltpu.sync_copy(zeros_hbm.at[pl.ds(s * RPT, RPT)], acc.at[pl.ds(s * RPT, RPT)])

    @pl.when(s == NS - 1)
    def _():
        pltpu.sync_copy(zeros_hbm.at[pl.ds(NS * RPT, RTAIL)],
                        acc.at[pl.ds(NS * RPT, RTAIL)])

    for m in range(NPRE):
        pltpu.async_copy(edges_hbm.at[wid, m], idx_v.at[m], sem_i.at[m])
    for m in range(NPRE):
        pltpu.make_async_copy(edges_hbm.at[wid, m], idx_v.at[m],
                              sem_i.at[m]).wait()
    plsc.subcore_barrier()
    for g0 in range(NPRE):
        pltpu.async_copy(hp_hbm.at[idx_v.at[g0, 0]], rows_v.at[g0],
                         sem_g.at[g0])
    pltpu.async_copy(edges_hbm.at[wid, NPRE], idx_v.at[NPRE], sem_i.at[NPRE])

    def body(g, carry):
        b = lax.rem(g, NBUF)
        bi = lax.rem(g, NIDX)
        pltpu.make_async_copy(hp_hbm.at[idx_v.at[0, 0]], rows_v.at[b],
                              sem_g.at[b]).wait()
        pltpu.async_copy(rows_v.at[b], acc.at[idx_v.at[bi, 1]], sem_s.at[b],
                         add=True)

        @pl.when(g + NPRE < NCHA)
        def _():
            gn = g + NPRE
            bn = lax.rem(gn, NBUF)

            @pl.when(g >= 1)
            def _():
                pltpu.make_async_copy(rows_v.at[bn], acc.at[idx_v.at[0, 1]],
                                      sem_s.at[bn]).wait()

            bin_ = lax.rem(gn, NIDX)
            pltpu.make_async_copy(edges_hbm.at[wid, 0], idx_v.at[bin_],
                                  sem_i.at[bin_]).wait()
            pltpu.async_copy(hp_hbm.at[idx_v.at[bin_, 0]], rows_v.at[bn],
                             sem_g.at[bn])

        @pl.when(g + NPRE + 1 < NCHA)
        def _():
            m = g + NPRE + 1
            bim = lax.rem(m, NIDX)
            pltpu.async_copy(edges_hbm.at[wid, m], idx_v.at[bim],
                             sem_i.at[bim])

        return carry

    lax.fori_loop(0, NCHA, body, 0)
    for t in range(NCHA - NBUF, NCHA):
        b = t % NBUF
        pltpu.make_async_copy(rows_v.at[b], acc.at[idx_v.at[0, 1]],
                              sem_s.at[b]).wait()
    plsc.subcore_barrier()
    pltpu.sync_copy(acc.at[pl.ds(s * RPT, RPT)], out_hbm.at[c, pl.ds(s * RPT, RPT)])

    @pl.when(s == NS - 1)
    def _():
        pltpu.sync_copy(acc.at[pl.ds(NS * RPT, RTAIL)],
                        out_hbm.at[c, pl.ds(NS * RPT, RTAIL)])


@functools.partial(
    pl.kernel,
    mesh=_mesh,
    out_type=jax.ShapeDtypeStruct((NC, N), jnp.float32),
    scratch_types=[
        pltpu.VMEM((NCHUNK, CH), jnp.int32),
        pltpu.VMEM((CH,), jnp.float32),
        pltpu.VMEM_SHARED((N,), jnp.float32),
    ],
)
def _sc_degree(dst_hbm, zeros_hbm, out_hbm, dst_v, ones_v, acc):
    c = lax.axis_index("c")
    s = lax.axis_index("s")
    wid = s * NC + c
    pltpu.sync_copy(dst_hbm.at[wid], dst_v)
    for j in range(CH // 16):
        ones_v[pl.ds(j * 16, 16)] = jnp.ones((16,), jnp.float32)

    @pl.when(s == 0)
    def _():
        pltpu.sync_copy(zeros_hbm, acc)

    plsc.subcore_barrier()

    def body(i, carry):
        pltpu.sync_copy(ones_v, acc.at[dst_v.at[i]], add=True)
        return carry

    lax.fori_loop(0, NCHUNK, body, 0)
    plsc.subcore_barrier()

    @pl.when(s == 0)
    def _():
        pltpu.sync_copy(acc, out_hbm.at[c])


def _dinv_of(deg_blk):
    return lax.rsqrt(jnp.sum(deg_blk, axis=1, keepdims=True) + 1.0)


def _tc_first_body(x_ref, w_ref, deg_ref, hp_ref):
    dinv = _dinv_of(deg_ref[...])
    hp_ref[...] = jnp.dot(x_ref[...], w_ref[...],
                          preferred_element_type=jnp.float32) * dinv


def _tc_first(x, w, deg8):
    return pl.pallas_call(
        _tc_first_body,
        grid=(NB,),
        in_specs=[
            pl.BlockSpec((BN, D), lambda i: (i, 0)),
            pl.BlockSpec((D, H), lambda i: (0, 0)),
            pl.BlockSpec((BN, 8), lambda i: (i, 0)),
        ],
        out_specs=pl.BlockSpec((BN, H), lambda i: (i, 0)),
        out_shape=jax.ShapeDtypeStruct((N, H), jnp.float32),
    )(x, w, deg8)


def _tc_mid_body(acc_ref, hp_ref, deg_ref, w_ref, p_ref, out_ref):
    dinv = _dinv_of(deg_ref[...])
    p = p_ref[...]
    b, g, be, rm, rv = p[0:1], p[1:2], p[2:3], p[3:4], p[4:5]
    t = (acc_ref[0] + acc_ref[1] + hp_ref[...]) * dinv + b
    t = (t - rm) * (g * lax.rsqrt(rv + 1e-5)) + be
    t = jnp.maximum(t, 0.0)
    out_ref[...] = jnp.dot(t, w_ref[...],
                           preferred_element_type=jnp.float32) * dinv


def _tc_mid(acc, hp, deg8, w, pvec):
    return pl.pallas_call(
        _tc_mid_body,
        grid=(NB,),
        in_specs=[
            pl.BlockSpec((NC, BN, H), lambda i: (0, i, 0)),
            pl.BlockSpec((BN, H), lambda i: (i, 0)),
            pl.BlockSpec((BN, 8), lambda i: (i, 0)),
            pl.BlockSpec((H, H), lambda i: (0, 0)),
            pl.BlockSpec((8, H), lambda i: (0, 0)),
        ],
        out_specs=pl.BlockSpec((BN, H), lambda i: (i, 0)),
        out_shape=jax.ShapeDtypeStruct((N, H), jnp.float32),
    )(acc, hp, deg8, w, pvec)


def _tc_final_body(acc_ref, hp_ref, deg_ref, b2_ref, batch_ref, lw_ref, lb_ref,
                   out_ref, sums, cnt):
    i = pl.program_id(0)

    @pl.when(i == 0)
    def _():
        sums[...] = jnp.zeros_like(sums)
        cnt[...] = jnp.zeros_like(cnt)

    dinv = _dinv_of(deg_ref[...])
    conv = (acc_ref[0] + acc_ref[1] + hp_ref[...]) * dinv + b2_ref[...]
    onehot = (batch_ref[...] ==
              lax.broadcasted_iota(jnp.int32, (BN, G), 1)).astype(jnp.float32)
    sums[...] += lax.dot_general(onehot, conv, (((0,), (0,)), ((), ())),
                                 preferred_element_type=jnp.float32)
    cnt[...] += lax.dot_general(onehot, jnp.ones((BN, H), jnp.float32),
                                (((0,), (0,)), ((), ())),
                                preferred_element_type=jnp.float32)

    @pl.when(i == pl.num_programs(0) - 1)
    def _():
        pooled = sums[...] / jnp.maximum(cnt[...], 1.0)
        out_ref[...] = jnp.dot(pooled, lw_ref[...],
                               preferred_element_type=jnp.float32) + lb_ref[...]


def _tc_final(acc, hp, deg8, b2, batch2d, lw, lb2d):
    return pl.pallas_call(
        _tc_final_body,
        grid=(NB,),
        in_specs=[
            pl.BlockSpec((NC, BN, H), lambda i: (0, i, 0)),
            pl.BlockSpec((BN, H), lambda i: (i, 0)),
            pl.BlockSpec((BN, 8), lambda i: (i, 0)),
            pl.BlockSpec((1, H), lambda i: (0, 0)),
            pl.BlockSpec((BN, 1), lambda i: (i, 0)),
            pl.BlockSpec((H, O), lambda i: (0, 0)),
            pl.BlockSpec((1, O), lambda i: (0, 0)),
        ],
        out_specs=pl.BlockSpec((G, O), lambda i: (0, 0)),
        out_shape=jax.ShapeDtypeStruct((G, O), jnp.float32),
        scratch_shapes=[
            pltpu.VMEM((G, H), jnp.float32),
            pltpu.VMEM((G, H), jnp.float32),
        ],
    )(acc, hp, deg8, b2, batch2d, lw, lb2d)


def kernel(x, edge_index, batch, W0, b0, W1, b1, W2, b2,
           g0, be0, rm0, rv0, g1, be1, rm1, rv1, lw, lb):
    src = edge_index[0].reshape(NW, NCHUNK, CH)
    dst = edge_index[1].reshape(NW, NCHUNK, CH)
    srcp = jnp.concatenate(
        [edge_index[0].reshape(NW, EPW),
         jnp.zeros((NW, NPAD), jnp.int32)], axis=1).reshape(NW, NCHA, CHA)
    dstp = jnp.concatenate(
        [edge_index[1].reshape(NW, EPW),
         jnp.full((NW, NPAD), N, jnp.int32)], axis=1).reshape(NW, NCHA, CHA)
    edges = jnp.stack([srcp, dstp], axis=2)
    zeros_nh = jnp.zeros((N, H), jnp.float32)
    zeros_n = jnp.zeros((N,), jnp.float32)

    deg2 = _sc_degree(dst, zeros_n)
    deg8 = jnp.concatenate(
        [jnp.swapaxes(deg2, 0, 1), jnp.zeros((N, 6), jnp.float32)], axis=1)

    p1 = jnp.concatenate([b0[None], g0[None], be0[None], rm0[None], rv0[None],
                          jnp.zeros((3, H), jnp.float32)], axis=0)
    p2 = jnp.concatenate([b1[None], g1[None], be1[None], rm1[None], rv1[None],
                          jnp.zeros((3, H), jnp.float32)], axis=0)

    hp0 = _tc_first(x, W0, deg8)
    acc0 = _sc_aggregate(hp0, edges, zeros_nh)
    hp1 = _tc_mid(acc0, hp0, deg8, W1, p1)
    acc1 = _sc_aggregate(hp1, edges, zeros_nh)
    hp2 = _tc_mid(acc1, hp1, deg8, W2, p2)
    acc2 = _sc_aggregate(hp2, edges, zeros_nh)
    return _tc_final(acc2, hp2, deg8, b2.reshape(1, H),
                     batch.reshape(N, 1), lw, lb.reshape(1, O))


# speedup vs baseline: 3.1876x; 3.1876x over previous
import functools

import jax
import jax.numpy as jnp
from jax import lax
from jax.experimental import pallas as pl
from jax.experimental.pallas import tpu as pltpu
from jax.experimental.pallas import tpu_sc as plsc

N = 10000
E = 320000
D = 128
H = 128
O = 64
G = 128

NC = 2
NS = 16
NW = NC * NS
CH = 80
EPW = E // NW
NCHUNK = EPW // CH
RPT = 624
RTAIL = N - NS * RPT

CHA = 80
EPWP = 10000
NCHA = EPWP // CHA
NPAD = EPWP - EPW
NA = N + 16

BN = 2000
NB = N // BN

_mesh = plsc.VectorSubcoreMesh(core_axis_name="c", subcore_axis_name="s")


NBUF = 4
NPRE = NBUF - 1
NIDX = NBUF + 1


@functools.partial(
    pl.kernel,
    mesh=_mesh,
    out_type=jax.ShapeDtypeStruct((NC, N, H), jnp.float32),
    scratch_types=[
        pltpu.VMEM((NIDX, 2, CHA), jnp.int32),
        pltpu.VMEM((NBUF, CHA, H), jnp.float32),
        pltpu.VMEM_SHARED((NA, H), jnp.float32),
        pltpu.SemaphoreType.DMA((NIDX,)),
        pltpu.SemaphoreType.DMA((NBUF,)),
        pltpu.SemaphoreType.DMA((NBUF,)),
    ],
)
def _sc_aggregate(hp_hbm, edges_hbm, zeros_hbm, out_hbm,
                  idx_v, rows_v, acc, sem_i, sem_g, sem_s):
    c = lax.axis_index("c")
    s = lax.axis_index("s")
    wid = s * NC + c
    pltpu.sync_copy(zeros_hbm.at[pl.ds(s * RPT, RPT)], acc.at[pl.ds(s * RPT, RPT)])

    @pl.when(s == NS - 1)
    def _():
        pltpu.sync_copy(zeros_hbm.at[pl.ds(NS * RPT, RTAIL)],
                        acc.at[pl.ds(NS * RPT, RTAIL)])

    for m in range(NPRE):
        pltpu.async_copy(edges_hbm.at[wid, m], idx_v.at[m], sem_i.at[m])
    for m in range(NPRE):
        pltpu.make_async_copy(edges_hbm.at[wid, m], idx_v.at[m],
                              sem_i.at[m]).wait()
    plsc.subcore_barrier()
    for g0 in range(NPRE):
        pltpu.async_copy(hp_hbm.at[idx_v.at[g0, 0]], rows_v.at[g0],
                         sem_g.at[g0])
    pltpu.async_copy(edges_hbm.at[wid, NPRE], idx_v.at[NPRE], sem_i.at[NPRE])

    def body(g, carry):
        b = lax.rem(g, NBUF)
        bi = lax.rem(g, NIDX)
        pltpu.make_async_copy(hp_hbm.at[idx_v.at[0, 0]], rows_v.at[b],
                              sem_g.at[b]).wait()
        pltpu.async_copy(rows_v.at[b], acc.at[idx_v.at[bi, 1]], sem_s.at[b],
                         add=True)

        @pl.when(g + NPRE < NCHA)
        def _():
            gn = g + NPRE
            bn = lax.rem(gn, NBUF)

            @pl.when(g >= 1)
            def _():
                pltpu.make_async_copy(rows_v.at[bn], acc.at[idx_v.at[0, 1]],
                                      sem_s.at[bn]).wait()

            bin_ = lax.rem(gn, NIDX)
            pltpu.make_async_copy(edges_hbm.at[wid, 0], idx_v.at[bin_],
                                  sem_i.at[bin_]).wait()
            pltpu.async_copy(hp_hbm.at[idx_v.at[bin_, 0]], rows_v.at[bn],
                             sem_g.at[bn])

        @pl.when(g + NPRE + 1 < NCHA)
        def _():
            m = g + NPRE + 1
            bim = lax.rem(m, NIDX)
            pltpu.async_copy(edges_hbm.at[wid, m], idx_v.at[bim],
                             sem_i.at[bim])

        return carry

    lax.fori_loop(0, NCHA, body, 0)
    for t in range(NCHA - NBUF, NCHA):
        b = t % NBUF
        pltpu.make_async_copy(rows_v.at[b], acc.at[idx_v.at[0, 1]],
                              sem_s.at[b]).wait()
    plsc.subcore_barrier()
    pltpu.sync_copy(acc.at[pl.ds(s * RPT, RPT)], out_hbm.at[c, pl.ds(s * RPT, RPT)])

    @pl.when(s == NS - 1)
    def _():
        pltpu.sync_copy(acc.at[pl.ds(NS * RPT, RTAIL)],
                        out_hbm.at[c, pl.ds(NS * RPT, RTAIL)])


@functools.partial(
    pl.kernel,
    mesh=_mesh,
    out_type=jax.ShapeDtypeStruct((NC, N), jnp.float32),
    scratch_types=[
        pltpu.VMEM((NCHUNK, CH), jnp.int32),
        pltpu.VMEM((CH,), jnp.float32),
        pltpu.VMEM_SHARED((N,), jnp.float32),
    ],
)
def _sc_degree(dst_hbm, zeros_hbm, out_hbm, dst_v, ones_v, acc):
    c = lax.axis_index("c")
    s = lax.axis_index("s")
    wid = s * NC + c
    pltpu.sync_copy(dst_hbm.at[wid], dst_v)
    for j in range(CH // 16):
        ones_v[pl.ds(j * 16, 16)] = jnp.ones((16,), jnp.float32)

    @pl.when(s == 0)
    def _():
        pltpu.sync_copy(zeros_hbm, acc)

    plsc.subcore_barrier()

    def body(i, carry):
        pltpu.sync_copy(ones_v, acc.at[dst_v.at[i]], add=True)
        return carry

    lax.fori_loop(0, NCHUNK, body, 0)
    plsc.subcore_barrier()

    @pl.when(s == 0)
    def _():
        pltpu.sync_copy(acc, out_hbm.at[c])


def _dinv_of(deg_blk):
    return lax.rsqrt(jnp.sum(deg_blk, axis=1, keepdims=True) + 1.0)


def _tc_first_body(x_ref, w_ref, deg_ref, hp_ref):
    dinv = _dinv_of(deg_ref[...])
    hp_ref[...] = jnp.dot(x_ref[...], w_ref[...],
                          preferred_element_type=jnp.float32) * dinv


def _tc_first(x, w, deg8):
    return pl.pallas_call(
        _tc_first_body,
        grid=(NB,),
        in_specs=[
            pl.BlockSpec((BN, D), lambda i: (i, 0)),
            pl.BlockSpec((D, H), lambda i: (0, 0)),
            pl.BlockSpec((BN, 8), lambda i: (i, 0)),
        ],
        out_specs=pl.BlockSpec((BN, H), lambda i: (i, 0)),
        out_shape=jax.ShapeDtypeStruct((N, H), jnp.float32),
    )(x, w, deg8)


def _tc_mid_body(acc_ref, hp_ref, deg_ref, w_ref, p_ref, out_ref):
    dinv = _dinv_of(deg_ref[...])
    p = p_ref[...]
    b, g, be, rm, rv = p[0:1], p[1:2], p[2:3], p[3:4], p[4:5]
    t = (acc_ref[0] + acc_ref[1] + hp_ref[...]) * dinv + b
    t = (t - rm) * (g * lax.rsqrt(rv + 1e-5)) + be
    t = jnp.maximum(t, 0.0)
    out_ref[...] = jnp.dot(t, w_ref[...],
                           preferred_element_type=jnp.float32) * dinv


def _tc_mid(acc, hp, deg8, w, pvec):
    return pl.pallas_call(
        _tc_mid_body,
        grid=(NB,),
        in_specs=[
            pl.BlockSpec((NC, BN, H), lambda i: (0, i, 0)),
            pl.BlockSpec((BN, H), lambda i: (i, 0)),
            pl.BlockSpec((BN, 8), lambda i: (i, 0)),
            pl.BlockSpec((H, H), lambda i: (0, 0)),
            pl.BlockSpec((8, H), lambda i: (0, 0)),
        ],
        out_specs=pl.BlockSpec((BN, H), lambda i: (i, 0)),
        out_shape=jax.ShapeDtypeStruct((N, H), jnp.float32),
    )(acc, hp, deg8, w, pvec)


def _tc_final_body(acc_ref, hp_ref, deg_ref, b2_ref, batch_ref, lw_ref, lb_ref,
                   out_ref, sums, cnt):
    i = pl.program_id(0)

    @pl.when(i == 0)
    def _():
        sums[...] = jnp.zeros_like(sums)
        cnt[...] = jnp.zeros_like(cnt)

    dinv = _dinv_of(deg_ref[...])
    conv = (acc_ref[0] + acc_ref[1] + hp_ref[...]) * dinv + b2_ref[...]
    onehot = (batch_ref[...] ==
              lax.broadcasted_iota(jnp.int32, (BN, G), 1)).astype(jnp.float32)
    sums[...] += lax.dot_general(onehot, conv, (((0,), (0,)), ((), ())),
                                 preferred_element_type=jnp.float32)
    cnt[...] += lax.dot_general(onehot, jnp.ones((BN, H), jnp.float32),
                                (((0,), (0,)), ((), ())),
                                preferred_element_type=jnp.float32)

    @pl.when(i == pl.num_programs(0) - 1)
    def _():
        pooled = sums[...] / jnp.maximum(cnt[...], 1.0)
        out_ref[...] = jnp.dot(pooled, lw_ref[...],
                               preferred_element_type=jnp.float32) + lb_ref[...]


def _tc_final(acc, hp, deg8, b2, batch2d, lw, lb2d):
    return pl.pallas_call(
        _tc_final_body,
        grid=(NB,),
        in_specs=[
            pl.BlockSpec((NC, BN, H), lambda i: (0, i, 0)),
            pl.BlockSpec((BN, H), lambda i: (i, 0)),
            pl.BlockSpec((BN, 8), lambda i: (i, 0)),
            pl.BlockSpec((1, H), lambda i: (0, 0)),
            pl.BlockSpec((BN, 1), lambda i: (i, 0)),
            pl.BlockSpec((H, O), lambda i: (0, 0)),
            pl.BlockSpec((1, O), lambda i: (0, 0)),
        ],
        out_specs=pl.BlockSpec((G, O), lambda i: (0, 0)),
        out_shape=jax.ShapeDtypeStruct((G, O), jnp.float32),
        scratch_shapes=[
            pltpu.VMEM((G, H), jnp.float32),
            pltpu.VMEM((G, H), jnp.float32),
        ],
    )(acc, hp, deg8, b2, batch2d, lw, lb2d)


def kernel(x, edge_index, batch, W0, b0, W1, b1, W2, b2,
           g0, be0, rm0, rv0, g1, be1, rm1, rv1, lw, lb):
    src = edge_index[0].reshape(NW, NCHUNK, CH)
    dst = edge_index[1].reshape(NW, NCHUNK, CH)
    srcp = jnp.concatenate(
        [edge_index[0].reshape(NW, EPW),
         jnp.zeros((NW, NPAD), jnp.int32)], axis=1).reshape(NW, NCHA, CHA)
    dstp = jnp.concatenate(
        [edge_index[1].reshape(NW, EPW),
         jnp.full((NW, NPAD), N, jnp.int32)], axis=1).reshape(NW, NCHA, CHA)
    edges = jnp.stack([srcp, dstp], axis=2)
    zeros_nh = jnp.zeros((N, H), jnp.float32)
    zeros_n = jnp.zeros((N,), jnp.float32)

    deg2 = _sc_degree(dst, zeros_n)
    deg8 = jnp.concatenate(
        [jnp.swapaxes(deg2, 0, 1), jnp.zeros((N, 6), jnp.float32)], axis=1)

    p1 = jnp.concatenate([b0[None], g0[None], be0[None], rm0[None], rv0[None],
                          jnp.zeros((3, H), jnp.float32)], axis=0)
    p2 = jnp.concatenate([b1[None], g1[None], be1[None], rm1[None], rv1[None],
                          jnp.zeros((3, H), jnp.float32)], axis=0)

    hp0 = _tc_first(x, W0, deg8)
    acc0 = _sc_aggregate(hp0, edges, zeros_nh)
    hp1 = _tc_mid(acc0, hp0, deg8, W1, p1)
    acc1 = _sc_aggregate(hp1, edges, zeros_nh)
    hp2 = _tc_mid(acc1, hp1, deg8, W2, p2)
    acc2 = _sc_aggregate(hp2, edges, zeros_nh)
    return _tc_final(acc2, hp2, deg8, b2.reshape(1, H),
                     batch.reshape(N, 1), lw, lb.reshape(1, O))


# speedup vs baseline: 3.2546x; 1.0210x over previous
import functools

import jax
import jax.numpy as jnp
from jax import lax
from jax.experimental import pallas as pl
from jax.experimental.pallas import tpu as pltpu
from jax.experimental.pallas import tpu_sc as plsc

N = 10000
E = 320000
D = 128
H = 128
O = 64
G = 128

NC = 2
NS = 16
NW = NC * NS
CH = 80
EPW = E // NW
NCHUNK = EPW // CH
RPT = 624
RTAIL = N - NS * RPT

CHA = 80
EPWP = 10000
NCHA = EPWP // CHA
NPAD = EPWP - EPW
NA = N + 16

BN = 2000
NB = N // BN

_mesh = plsc.VectorSubcoreMesh(core_axis_name="c", subcore_axis_name="s")


NBUF = 4
NPRE = NBUF - 1
NIDX = NBUF + 1


@functools.partial(
    pl.kernel,
    mesh=_mesh,
    out_type=jax.ShapeDtypeStruct((NC, N, H), jnp.float32),
    scratch_types=[
        pltpu.VMEM((NIDX, 2, CHA), jnp.int32),
        pltpu.VMEM((NBUF, CHA, H), jnp.float32),
        pltpu.VMEM_SHARED((NA, H), jnp.float32),
        pltpu.SemaphoreType.DMA((NIDX,)),
        pltpu.SemaphoreType.DMA((NBUF,)),
        pltpu.SemaphoreType.DMA((NBUF,)),
        pltpu.SemaphoreType.DMA((2,)),
    ],
)
def _sc_aggregate(hp_hbm, edges_hbm, zeros_hbm, out_hbm,
                  idx_v, rows_v, acc, sem_i, sem_g, sem_s, sem_z):
    c = lax.axis_index("c")
    s = lax.axis_index("s")
    wid = s * NC + c
    pltpu.async_copy(zeros_hbm.at[pl.ds(s * RPT, RPT)],
                     acc.at[pl.ds(s * RPT, RPT)], sem_z.at[0])

    @pl.when(s == NS - 1)
    def _():
        pltpu.async_copy(zeros_hbm.at[pl.ds(NS * RPT, RTAIL)],
                         acc.at[pl.ds(NS * RPT, RTAIL)], sem_z.at[1])

    for m in range(NPRE):
        pltpu.async_copy(edges_hbm.at[wid, m], idx_v.at[m], sem_i.at[m])
    for m in range(NPRE):
        pltpu.make_async_copy(edges_hbm.at[wid, m], idx_v.at[m],
                              sem_i.at[m]).wait()
    for g0 in range(NPRE):
        pltpu.async_copy(hp_hbm.at[idx_v.at[g0, 0]], rows_v.at[g0],
                         sem_g.at[g0])
    pltpu.async_copy(edges_hbm.at[wid, NPRE], idx_v.at[NPRE], sem_i.at[NPRE])
    pltpu.make_async_copy(zeros_hbm.at[pl.ds(s * RPT, RPT)],
                          acc.at[pl.ds(s * RPT, RPT)], sem_z.at[0]).wait()

    @pl.when(s == NS - 1)
    def _():
        pltpu.make_async_copy(zeros_hbm.at[pl.ds(NS * RPT, RTAIL)],
                              acc.at[pl.ds(NS * RPT, RTAIL)], sem_z.at[1]).wait()

    plsc.subcore_barrier()

    def body(g, carry):
        b = lax.rem(g, NBUF)
        bi = lax.rem(g, NIDX)
        pltpu.make_async_copy(hp_hbm.at[idx_v.at[0, 0]], rows_v.at[b],
                              sem_g.at[b]).wait()
        pltpu.async_copy(rows_v.at[b], acc.at[idx_v.at[bi, 1]], sem_s.at[b],
                         add=True)

        @pl.when(g + NPRE < NCHA)
        def _():
            gn = g + NPRE
            bn = lax.rem(gn, NBUF)

            @pl.when(g >= 1)
            def _():
                pltpu.make_async_copy(rows_v.at[bn], acc.at[idx_v.at[0, 1]],
                                      sem_s.at[bn]).wait()

            bin_ = lax.rem(gn, NIDX)
            pltpu.make_async_copy(edges_hbm.at[wid, 0], idx_v.at[bin_],
                                  sem_i.at[bin_]).wait()
            pltpu.async_copy(hp_hbm.at[idx_v.at[bin_, 0]], rows_v.at[bn],
                             sem_g.at[bn])

        @pl.when(g + NPRE + 1 < NCHA)
        def _():
            m = g + NPRE + 1
            bim = lax.rem(m, NIDX)
            pltpu.async_copy(edges_hbm.at[wid, m], idx_v.at[bim],
                             sem_i.at[bim])

        return carry

    lax.fori_loop(0, NCHA, body, 0)
    for t in range(NCHA - NBUF, NCHA):
        b = t % NBUF
        pltpu.make_async_copy(rows_v.at[b], acc.at[idx_v.at[0, 1]],
                              sem_s.at[b]).wait()
    plsc.subcore_barrier()
    pltpu.sync_copy(acc.at[pl.ds(s * RPT, RPT)], out_hbm.at[c, pl.ds(s * RPT, RPT)])

    @pl.when(s == NS - 1)
    def _():
        pltpu.sync_copy(acc.at[pl.ds(NS * RPT, RTAIL)],
                        out_hbm.at[c, pl.ds(NS * RPT, RTAIL)])


@functools.partial(
    pl.kernel,
    mesh=_mesh,
    out_type=jax.ShapeDtypeStruct((NC, N), jnp.float32),
    scratch_types=[
        pltpu.VMEM((NCHUNK, CH), jnp.int32),
        pltpu.VMEM((CH,), jnp.float32),
        pltpu.VMEM_SHARED((N,), jnp.float32),
    ],
)
def _sc_degree(dst_hbm, zeros_hbm, out_hbm, dst_v, ones_v, acc):
    c = lax.axis_index("c")
    s = lax.axis_index("s")
    wid = s * NC + c
    pltpu.sync_copy(dst_hbm.at[wid], dst_v)
    for j in range(CH // 16):
        ones_v[pl.ds(j * 16, 16)] = jnp.ones((16,), jnp.float32)

    @pl.when(s == 0)
    def _():
        pltpu.sync_copy(zeros_hbm, acc)

    plsc.subcore_barrier()

    def body(i, carry):
        pltpu.sync_copy(ones_v, acc.at[dst_v.at[i]], add=True)
        return carry

    lax.fori_loop(0, NCHUNK, body, 0)
    plsc.subcore_barrier()

    @pl.when(s == 0)
    def _():
        pltpu.sync_copy(acc, out_hbm.at[c])


def _dinv_of(deg_blk):
    return lax.rsqrt(jnp.sum(deg_blk, axis=1, keepdims=True) + 1.0)


def _tc_first_body(x_ref, w_ref, deg_ref, hp_ref):
    dinv = _dinv_of(deg_ref[...])
    hp_ref[...] = jnp.dot(x_ref[...], w_ref[...],
                          preferred_element_type=jnp.float32) * dinv


def _tc_first(x, w, deg8):
    return pl.pallas_call(
        _tc_first_body,
        grid=(NB,),
        in_specs=[
            pl.BlockSpec((BN, D), lambda i: (i, 0)),
            pl.BlockSpec((D, H), lambda i: (0, 0)),
            pl.BlockSpec((BN, 8), lambda i: (i, 0)),
        ],
        out_specs=pl.BlockSpec((BN, H), lambda i: (i, 0)),
        out_shape=jax.ShapeDtypeStruct((N, H), jnp.float32),
    )(x, w, deg8)


def _tc_mid_body(acc_ref, hp_ref, deg_ref, w_ref, p_ref, out_ref):
    dinv = _dinv_of(deg_ref[...])
    p = p_ref[...]
    b, g, be, rm, rv = p[0:1], p[1:2], p[2:3], p[3:4], p[4:5]
    t = (acc_ref[0] + acc_ref[1] + hp_ref[...]) * dinv + b
    t = (t - rm) * (g * lax.rsqrt(rv + 1e-5)) + be
    t = jnp.maximum(t, 0.0)
    out_ref[...] = jnp.dot(t, w_ref[...],
                           preferred_element_type=jnp.float32) * dinv


def _tc_mid(acc, hp, deg8, w, pvec):
    return pl.pallas_call(
        _tc_mid_body,
        grid=(NB,),
        in_specs=[
            pl.BlockSpec((NC, BN, H), lambda i: (0, i, 0)),
            pl.BlockSpec((BN, H), lambda i: (i, 0)),
            pl.BlockSpec((BN, 8), lambda i: (i, 0)),
            pl.BlockSpec((H, H), lambda i: (0, 0)),
            pl.BlockSpec((8, H), lambda i: (0, 0)),
        ],
        out_specs=pl.BlockSpec((BN, H), lambda i: (i, 0)),
        out_shape=jax.ShapeDtypeStruct((N, H), jnp.float32),
    )(acc, hp, deg8, w, pvec)


def _tc_final_body(acc_ref, hp_ref, deg_ref, b2_ref, batch_ref, lw_ref, lb_ref,
                   out_ref, sums, cnt):
    i = pl.program_id(0)

    @pl.when(i == 0)
    def _():
        sums[...] = jnp.zeros_like(sums)
        cnt[...] = jnp.zeros_like(cnt)

    dinv = _dinv_of(deg_ref[...])
    conv = (acc_ref[0] + acc_ref[1] + hp_ref[...]) * dinv + b2_ref[...]
    onehot = (batch_ref[...] ==
              lax.broadcasted_iota(jnp.int32, (BN, G), 1)).astype(jnp.float32)
    sums[...] += lax.dot_general(onehot, conv, (((0,), (0,)), ((), ())),
                                 preferred_element_type=jnp.float32)
    cnt[...] += lax.dot_general(onehot, jnp.ones((BN, H), jnp.float32),
                                (((0,), (0,)), ((), ())),
                                preferred_element_type=jnp.float32)

    @pl.when(i == pl.num_programs(0) - 1)
    def _():
        pooled = sums[...] / jnp.maximum(cnt[...], 1.0)
        out_ref[...] = jnp.dot(pooled, lw_ref[...],
                               preferred_element_type=jnp.float32) + lb_ref[...]


def _tc_final(acc, hp, deg8, b2, batch2d, lw, lb2d):
    return pl.pallas_call(
        _tc_final_body,
        grid=(NB,),
        in_specs=[
            pl.BlockSpec((NC, BN, H), lambda i: (0, i, 0)),
            pl.BlockSpec((BN, H), lambda i: (i, 0)),
            pl.BlockSpec((BN, 8), lambda i: (i, 0)),
            pl.BlockSpec((1, H), lambda i: (0, 0)),
            pl.BlockSpec((BN, 1), lambda i: (i, 0)),
            pl.BlockSpec((H, O), lambda i: (0, 0)),
            pl.BlockSpec((1, O), lambda i: (0, 0)),
        ],
        out_specs=pl.BlockSpec((G, O), lambda i: (0, 0)),
        out_shape=jax.ShapeDtypeStruct((G, O), jnp.float32),
        scratch_shapes=[
            pltpu.VMEM((G, H), jnp.float32),
            pltpu.VMEM((G, H), jnp.float32),
        ],
    )(acc, hp, deg8, b2, batch2d, lw, lb2d)


def kernel(x, edge_index, batch, W0, b0, W1, b1, W2, b2,
           g0, be0, rm0, rv0, g1, be1, rm1, rv1, lw, lb):
    src = edge_index[0].reshape(NW, NCHUNK, CH)
    dst = edge_index[1].reshape(NW, NCHUNK, CH)
    srcp = jnp.concatenate(
        [edge_index[0].reshape(NW, EPW),
         jnp.zeros((NW, NPAD), jnp.int32)], axis=1).reshape(NW, NCHA, CHA)
    dstp = jnp.concatenate(
        [edge_index[1].reshape(NW, EPW),
         jnp.full((NW, NPAD), N, jnp.int32)], axis=1).reshape(NW, NCHA, CHA)
    edges = jnp.stack([srcp, dstp], axis=2)
    zeros_nh = jnp.zeros((N, H), jnp.float32)
    zeros_n = jnp.zeros((N,), jnp.float32)

    deg2 = _sc_degree(dst, zeros_n)
    deg8 = jnp.concatenate(
        [jnp.swapaxes(deg2, 0, 1), jnp.zeros((N, 6), jnp.float32)], axis=1)

    p1 = jnp.concatenate([b0[None], g0[None], be0[None], rm0[None], rv0[None],
                          jnp.zeros((3, H), jnp.float32)], axis=0)
    p2 = jnp.concatenate([b1[None], g1[None], be1[None], rm1[None], rv1[None],
                          jnp.zeros((3, H), jnp.float32)], axis=0)

    hp0 = _tc_first(x, W0, deg8)
    acc0 = _sc_aggregate(hp0, edges, zeros_nh)
    hp1 = _tc_mid(acc0, hp0, deg8, W1, p1)
    acc1 = _sc_aggregate(hp1, edges, zeros_nh)
    hp2 = _tc_mid(acc1, hp1, deg8, W2, p2)
    acc2 = _sc_aggregate(hp2, edges, zeros_nh)
    return _tc_final(acc2, hp2, deg8, b2.reshape(1, H),
                     batch.reshape(N, 1), lw, lb.reshape(1, O))
